# baseline (speedup 1.0000x reference)
import jax
import jax.numpy as jnp
from jax.experimental import pallas as pl
from jax.experimental.pallas import tpu as pltpu

_GRID_SCALE = 0.0001
_FREQS = (1.0, 2.0, 4.0, 8.0, 16.0)
_BLOCK = 2000


def _mlp_block(v_ref, w0, w1, w2, w3, w4, w5, w6, w7, w8, out_ref):
    p = v_ref[...]
    feats = [p]
    for f in _FREQS:
        pf = p * f
        feats.append(jnp.sin(pf))
        feats.append(jnp.cos(pf))
    e = jnp.concatenate(feats, axis=1)
    h = jnp.maximum(jnp.dot(e, w0[...], preferred_element_type=jnp.float32), 0.0)
    for w in (w1, w2, w3, w4, w5, w6, w7):
        h = jnp.maximum(jnp.dot(h, w[...], preferred_element_type=jnp.float32), 0.0)
    out = jnp.dot(h, w8[...], preferred_element_type=jnp.float32)
    sdf = out[:, 0:1]
    v_def = p + _GRID_SCALE * jnp.tanh(out[:, 1:4])
    out_ref[...] = jnp.concatenate([sdf, v_def], axis=1)


def kernel(vertices, indices, W0, W1, W2, W3, W4, W5, W6, W7, W8):
    del indices
    n = vertices.shape[0]
    grid = n // _BLOCK

    def w_spec(w):
        return pl.BlockSpec(w.shape, lambda i: (0, 0))

    return pl.pallas_call(
        _mlp_block,
        grid=(grid,),
        in_specs=[
            pl.BlockSpec((_BLOCK, 3), lambda i: (i, 0)),
            w_spec(W0), w_spec(W1), w_spec(W2), w_spec(W3), w_spec(W4),
            w_spec(W5), w_spec(W6), w_spec(W7), w_spec(W8),
        ],
        out_specs=pl.BlockSpec((_BLOCK, 4), lambda i: (i, 0)),
        out_shape=jax.ShapeDtypeStruct((n, 4), jnp.float32),
    )(vertices, W0, W1, W2, W3, W4, W5, W6, W7, W8)


# speedup vs baseline: 12.6694x; 12.6694x over previous
import jax
import jax.numpy as jnp
from jax.experimental import pallas as pl

_GRID_SCALE = 0.0001
_BLOCK = 4096


def _mlp_block(vt_ref, w0t, w1t, w2t, w3t, w4t, w5t, w6t, w7t, w8t, out_ref):
    p = vt_ref[...]
    s1 = jnp.sin(p)
    c1 = jnp.cos(p)
    s2 = 2.0 * s1 * c1
    c2 = 1.0 - 2.0 * s1 * s1
    s4 = 2.0 * s2 * c2
    c4 = 1.0 - 2.0 * s2 * s2
    s8 = 2.0 * s4 * c4
    c8 = 1.0 - 2.0 * s4 * s4
    s16 = 2.0 * s8 * c8
    c16 = 1.0 - 2.0 * s8 * s8
    e = jnp.concatenate(
        [p, s1, c1, s2, c2, s4, c4, s8, c8, s16, c16], axis=0)
    h = jnp.maximum(jnp.dot(w0t[...], e, preferred_element_type=jnp.float32), 0.0)
    for wt in (w1t, w2t, w3t, w4t, w5t, w6t, w7t):
        h = jnp.maximum(jnp.dot(wt[...], h, preferred_element_type=jnp.float32), 0.0)
    out = jnp.dot(w8t[...], h, preferred_element_type=jnp.float32)
    v_def = p + _GRID_SCALE * jnp.tanh(out[1:4, :])
    out_ref[...] = jnp.concatenate([out[0:1, :], v_def], axis=0)


def kernel(vertices, indices, W0, W1, W2, W3, W4, W5, W6, W7, W8):
    del indices
    n = vertices.shape[0]
    grid = (n + _BLOCK - 1) // _BLOCK
    vt = vertices.T
    wts = [w.T for w in (W0, W1, W2, W3, W4, W5, W6, W7, W8)]

    def w_spec(w):
        return pl.BlockSpec(w.shape, lambda i: (0, 0))

    out_t = pl.pallas_call(
        _mlp_block,
        grid=(grid,),
        in_specs=[pl.BlockSpec((3, _BLOCK), lambda i: (0, i))]
        + [w_spec(w) for w in wts],
        out_specs=pl.BlockSpec((4, _BLOCK), lambda i: (0, i)),
        out_shape=jax.ShapeDtypeStruct((4, n), jnp.float32),
    )(vt, *wts)
    return out_t.T
